# baseline (speedup 1.0000x reference)
import jax
import jax.numpy as jnp
from jax.experimental import pallas as pl
from jax.experimental.pallas import tpu as pltpu

_GRID = 12
_LANES = 512
_MAIN_ROWS = (16 * 3 * 96 * 320) // _LANES
_VERT_ROWS = (16 * 9 * 96 * 320) // _LANES
_MAIN_BLK = _MAIN_ROWS // _GRID
_VERT_BLK = _VERT_ROWS // _GRID


def _focal_partial(x, t):
    pred = jnp.clip(jax.nn.sigmoid(x), 1e-4, 1.0 - 1e-4)
    pos = (t >= 0.9999).astype(jnp.float32)
    neg = 1.0 - pos
    one_m_pred = 1.0 - pred
    pos_loss = jnp.log(pred) * (one_m_pred * one_m_pred) * pos
    omt = 1.0 - t
    omt2 = omt * omt
    neg_loss = jnp.log(one_m_pred) * (pred * pred) * (omt2 * omt2) * neg
    return jnp.sum(pos_loss + neg_loss), jnp.sum(pos)


def _body(ml_ref, mm_ref, vl_ref, vm_ref, out_ref, acc_ref):
    i = pl.program_id(0)
    ms, mp = _focal_partial(ml_ref[...], mm_ref[...])
    vs, vp = _focal_partial(vl_ref[...], vm_ref[...])

    @pl.when(i == 0)
    def _init():
        acc_ref[0] = ms
        acc_ref[1] = mp
        acc_ref[2] = vs
        acc_ref[3] = vp

    @pl.when(i > 0)
    def _accum():
        acc_ref[0] += ms
        acc_ref[1] += mp
        acc_ref[2] += vs
        acc_ref[3] += vp

    @pl.when(i == _GRID - 1)
    def _finalize():
        main_loss = acc_ref[0] / jnp.maximum(acc_ref[1], 1.0)
        vert_loss = acc_ref[2] / jnp.maximum(acc_ref[3], 1.0)
        out_ref[0, 0] = -(main_loss + vert_loss)


def kernel(main_kf_logits, main_kf_mask, vertex_kf_logits, vertex_kf_mask):
    ml = main_kf_logits.reshape(_MAIN_ROWS, _LANES)
    mm = main_kf_mask.reshape(_MAIN_ROWS, _LANES)
    vl = vertex_kf_logits.reshape(_VERT_ROWS, _LANES)
    vm = vertex_kf_mask.reshape(_VERT_ROWS, _LANES)

    main_spec = pl.BlockSpec((_MAIN_BLK, _LANES), lambda i: (i, 0))
    vert_spec = pl.BlockSpec((_VERT_BLK, _LANES), lambda i: (i, 0))

    out = pl.pallas_call(
        _body,
        grid=(_GRID,),
        in_specs=[main_spec, main_spec, vert_spec, vert_spec],
        out_specs=pl.BlockSpec(memory_space=pltpu.SMEM),
        out_shape=jax.ShapeDtypeStruct((1, 1), jnp.float32),
        scratch_shapes=[pltpu.SMEM((4,), jnp.float32)],
        compiler_params=pltpu.CompilerParams(
            dimension_semantics=("arbitrary",),
        ),
    )(ml, mm, vl, vm)
    return out[0, 0]
